# speedup vs baseline: 8.2788x; 8.2788x over previous
import functools

import jax
import jax.numpy as jnp
from jax import lax
from jax.experimental import pallas as pl
from jax.experimental.pallas import tpu as pltpu

N = 10000
E = 320000
F = 128
HP = 16
ROWS = 2000
NB = N // ROWS
NEG = -1e30


def _head_matrix(h):
    col = lax.broadcasted_iota(jnp.int32, (F, HP), 1)
    row = lax.broadcasted_iota(jnp.int32, (F, HP), 0)
    d = F // h
    return (row // d == col).astype(jnp.float32)


def _dense_stage_kernel(h, x_ref, w_ref, alf_ref, arf_ref,
                        ft_ref, elp_ref, erp_ref, elmax_ref, ermax_ref):
    i = pl.program_id(0)
    x = x_ref[...]
    ft = lax.dot_general(x, w_ref[...], (((1,), (1,)), ((), ())),
                         preferred_element_type=jnp.float32)
    ft_ref[...] = ft
    sm = _head_matrix(h)
    el = jnp.dot(ft * alf_ref[...], sm, preferred_element_type=jnp.float32)
    er = jnp.dot(ft * arf_ref[...], sm, preferred_element_type=jnp.float32)
    lane = lax.broadcasted_iota(jnp.int32, (ROWS, HP), 1)
    elp = jnp.where(lane < h, el, NEG)
    erp = jnp.where(lane < h, er, NEG)
    elp_ref[...] = elp
    erp_ref[...] = erp
    bmax_l = jnp.max(elp, axis=0, keepdims=True)
    bmax_r = jnp.max(erp, axis=0, keepdims=True)

    @pl.when(i == 0)
    def _():
        elmax_ref[...] = bmax_l
        ermax_ref[...] = bmax_r

    @pl.when(i > 0)
    def _():
        elmax_ref[...] = jnp.maximum(elmax_ref[...], bmax_l)
        ermax_ref[...] = jnp.maximum(ermax_ref[...], bmax_r)


def _dense_stage(x, w, alf, arf, h):
    f32 = jnp.float32
    return pl.pallas_call(
        functools.partial(_dense_stage_kernel, h),
        grid=(NB,),
        in_specs=[
            pl.BlockSpec((ROWS, F), lambda i: (i, 0)),
            pl.BlockSpec((F, F), lambda i: (0, 0)),
            pl.BlockSpec((1, F), lambda i: (0, 0)),
            pl.BlockSpec((1, F), lambda i: (0, 0)),
        ],
        out_specs=[
            pl.BlockSpec((ROWS, F), lambda i: (i, 0)),
            pl.BlockSpec((ROWS, HP), lambda i: (i, 0)),
            pl.BlockSpec((ROWS, HP), lambda i: (i, 0)),
            pl.BlockSpec((1, HP), lambda i: (0, 0)),
            pl.BlockSpec((1, HP), lambda i: (0, 0)),
        ],
        out_shape=[
            jax.ShapeDtypeStruct((N, F), f32),
            jax.ShapeDtypeStruct((N, HP), f32),
            jax.ShapeDtypeStruct((N, HP), f32),
            jax.ShapeDtypeStruct((1, HP), f32),
            jax.ShapeDtypeStruct((1, HP), f32),
        ],
    )(x, w, alf, arf)


def _combine_kernel(h, num_ref, den_ref, out_ref):
    num = num_ref[0] + num_ref[1]
    den = den_ref[0] + den_ref[1]
    den = jnp.where(den == 0.0, 1.0, den)
    d = F // h
    parts = []
    for g in range(h):
        r = 1.0 / den[:, g:g + 1]
        parts.append(num[:, g * d:(g + 1) * d] * r)
    out_ref[...] = jnp.concatenate(parts, axis=1)


def _combine(num2, den2, h):
    return pl.pallas_call(
        functools.partial(_combine_kernel, h),
        grid=(NB,),
        in_specs=[
            pl.BlockSpec((2, ROWS, F), lambda i: (0, i, 0)),
            pl.BlockSpec((2, ROWS, HP), lambda i: (0, i, 0)),
        ],
        out_specs=pl.BlockSpec((ROWS, F), lambda i: (i, 0)),
        out_shape=jax.ShapeDtypeStruct((N, F), jnp.float32),
    )(num2, den2)


def _edge_stage(ft, elp, erp, elmax, ermax, src, dst, h):
    m = jnp.maximum(elmax + ermax, 0.2 * (elmax + ermax))
    e = elp[src] + erp[dst]
    e = jnp.maximum(e, 0.2 * e)
    ee = jnp.exp(e - m)
    d = F // h
    coef = jnp.repeat(ee[:, :h], d, axis=1)
    num = jax.ops.segment_sum(ft[src] * coef, dst, num_segments=N)
    den = jax.ops.segment_sum(ee, dst, num_segments=N)
    return jnp.stack([num, jnp.zeros_like(num)]), \
        jnp.stack([den, jnp.zeros_like(den)])


def kernel(feats, g, W0, al0, ar0, W1, al1, ar1):
    src = g[0]
    dst = g[1]
    alf0 = al0.reshape(1, F)
    arf0 = ar0.reshape(1, F)
    alf1 = al1.reshape(1, F)
    arf1 = ar1.reshape(1, F)

    ft0, elp0, erp0, elm0, erm0 = _dense_stage(feats, W0, alf0, arf0, 8)
    num0, den0 = _edge_stage(ft0, elp0, erp0, elm0, erm0, src, dst, 8)
    h1 = _combine(num0, den0, 8)

    ft1, elp1, erp1, elm1, erm1 = _dense_stage(h1, W1, alf1, arf1, 1)
    num1, den1 = _edge_stage(ft1, elp1, erp1, elm1, erm1, src, dst, 1)
    h_final = _combine(num1, den1, 1)
    return (h1, h_final)


# speedup vs baseline: 37.6739x; 4.5507x over previous
import functools

import jax
import jax.numpy as jnp
from jax import lax
from jax.experimental import pallas as pl
from jax.experimental.pallas import tpu as pltpu
from jax.experimental.pallas import tpu_sc as plsc

N = 10000
E = 320000
F = 128
HP = 16
ROWS = 2000
NB = N // ROWS
NEG = -1e30


def _head_matrix(h):
    col = lax.broadcasted_iota(jnp.int32, (F, HP), 1)
    row = lax.broadcasted_iota(jnp.int32, (F, HP), 0)
    d = F // h
    return (row // d == col).astype(jnp.float32)


def _dense_stage_kernel(h, x_ref, w_ref, alf_ref, arf_ref,
                        ft_ref, elp_ref, erp_ref, elmax_ref, ermax_ref):
    i = pl.program_id(0)
    x = x_ref[...]
    ft = lax.dot_general(x, w_ref[...], (((1,), (1,)), ((), ())),
                         preferred_element_type=jnp.float32)
    ft_ref[...] = ft
    sm = _head_matrix(h)
    el = jnp.dot(ft * alf_ref[...], sm, preferred_element_type=jnp.float32)
    er = jnp.dot(ft * arf_ref[...], sm, preferred_element_type=jnp.float32)
    lane = lax.broadcasted_iota(jnp.int32, (ROWS, HP), 1)
    elp = jnp.where(lane < h, el, NEG)
    erp = jnp.where(lane < h, er, NEG)
    elp_ref[...] = elp
    erp_ref[...] = erp
    bmax_l = jnp.max(elp, axis=0, keepdims=True)
    bmax_r = jnp.max(erp, axis=0, keepdims=True)

    @pl.when(i == 0)
    def _():
        elmax_ref[...] = bmax_l
        ermax_ref[...] = bmax_r

    @pl.when(i > 0)
    def _():
        elmax_ref[...] = jnp.maximum(elmax_ref[...], bmax_l)
        ermax_ref[...] = jnp.maximum(ermax_ref[...], bmax_r)


def _dense_stage(x, w, alf, arf, h):
    f32 = jnp.float32
    return pl.pallas_call(
        functools.partial(_dense_stage_kernel, h),
        grid=(NB,),
        in_specs=[
            pl.BlockSpec((ROWS, F), lambda i: (i, 0)),
            pl.BlockSpec((F, F), lambda i: (0, 0)),
            pl.BlockSpec((1, F), lambda i: (0, 0)),
            pl.BlockSpec((1, F), lambda i: (0, 0)),
        ],
        out_specs=[
            pl.BlockSpec((ROWS, F), lambda i: (i, 0)),
            pl.BlockSpec((ROWS, HP), lambda i: (i, 0)),
            pl.BlockSpec((ROWS, HP), lambda i: (i, 0)),
            pl.BlockSpec((1, HP), lambda i: (0, 0)),
            pl.BlockSpec((1, HP), lambda i: (0, 0)),
        ],
        out_shape=[
            jax.ShapeDtypeStruct((N, F), f32),
            jax.ShapeDtypeStruct((N, HP), f32),
            jax.ShapeDtypeStruct((N, HP), f32),
            jax.ShapeDtypeStruct((1, HP), f32),
            jax.ShapeDtypeStruct((1, HP), f32),
        ],
    )(x, w, alf, arf)


def _combine_kernel(h, num_ref, den_ref, out_ref):
    num = num_ref[0] + num_ref[1]
    den = den_ref[0] + den_ref[1]
    den = jnp.where(den == 0.0, 1.0, den)
    d = F // h
    parts = []
    for g in range(h):
        r = 1.0 / den[:, g:g + 1]
        parts.append(num[:, g * d:(g + 1) * d] * r)
    out_ref[...] = jnp.concatenate(parts, axis=1)


def _combine(num2, den2, h):
    return pl.pallas_call(
        functools.partial(_combine_kernel, h),
        grid=(NB,),
        in_specs=[
            pl.BlockSpec((2, ROWS, F), lambda i: (0, i, 0)),
            pl.BlockSpec((2, ROWS, HP), lambda i: (0, i, 0)),
        ],
        out_specs=pl.BlockSpec((ROWS, F), lambda i: (i, 0)),
        out_shape=jax.ShapeDtypeStruct((N, F), jnp.float32),
    )(num2, den2)


NCORE = 2
NTILE = 16
NW = NCORE * NTILE
CB = 128
NCHUNK = E // CB
NP = 10240
RPT = NP // NTILE


def _edge_sc_body(head_of_group,
                  ft_hbm, elp_hbm, erp_hbm, elmax_hbm, ermax_hbm,
                  src_hbm, dst_hbm, z128_hbm, z16_hbm,
                  num_hbm, den_hbm,
                  sidx_v, didx_v, elb_v, erb_v, ftb_v, m_v,
                  num_sp, den_sp):
    c = lax.axis_index("c")
    s = lax.axis_index("s")
    wid = s * NCORE + c

    pltpu.sync_copy(z128_hbm.at[pl.ds(s * RPT, RPT)],
                    num_sp.at[pl.ds(s * RPT, RPT)])
    pltpu.sync_copy(z16_hbm.at[pl.ds(s * RPT, RPT)],
                    den_sp.at[pl.ds(s * RPT, RPT)])

    pltpu.sync_copy(elmax_hbm, m_v.at[pl.ds(0, 1)])
    pltpu.sync_copy(ermax_hbm, m_v.at[pl.ds(1, 1)])
    msum = m_v[0, :] + m_v[1, :]
    m_v[0, :] = jnp.maximum(msum, 0.2 * msum)

    plsc.subcore_barrier()

    @pl.loop(wid, NCHUNK, step=NW)
    def _(ch):
        base = ch * CB
        pltpu.sync_copy(src_hbm.at[pl.ds(base, CB)], sidx_v)
        pltpu.sync_copy(dst_hbm.at[pl.ds(base, CB)], didx_v)
        pltpu.sync_copy(ft_hbm.at[sidx_v], ftb_v)
        pltpu.sync_copy(elp_hbm.at[sidx_v], elb_v)
        pltpu.sync_copy(erp_hbm.at[didx_v], erb_v)
        mvec = m_v[0, :]

        @pl.loop(0, CB)
        def _(e):
            x = elb_v[e, :] + erb_v[e, :]
            x = jnp.maximum(x, 0.2 * x)
            ee = jnp.exp(x - mvec)
            elb_v[e, :] = ee
            dn = lax.GatherDimensionNumbers(
                offset_dims=(), collapsed_slice_dims=(0,),
                start_index_map=(0,))
            for g in range(8):
                hg = head_of_group[g]
                idx = jnp.full((HP, 1), hg, jnp.int32)
                coef = lax.gather(
                    ee, idx, dn, slice_sizes=(1,),
                    mode=lax.GatherScatterMode.PROMISE_IN_BOUNDS)
                sl = pl.ds(g * 16, 16)
                ftb_v[e, sl] = ftb_v[e, sl] * coef

        pltpu.sync_copy(ftb_v, num_sp.at[didx_v], add=True)
        pltpu.sync_copy(elb_v, den_sp.at[didx_v], add=True)

    plsc.subcore_barrier()
    pltpu.sync_copy(num_sp.at[pl.ds(s * RPT, RPT)],
                    num_hbm.at[c, pl.ds(s * RPT, RPT)])
    pltpu.sync_copy(den_sp.at[pl.ds(s * RPT, RPT)],
                    den_hbm.at[c, pl.ds(s * RPT, RPT)])


def _edge_stage(ft, elp, erp, elmax, ermax, src, dst, h):
    f32 = jnp.float32
    head_of_group = tuple(g if h == 8 else 0 for g in range(8))
    z128 = jnp.zeros((NP, F), f32)
    z16 = jnp.zeros((NP, HP), f32)
    mesh = plsc.VectorSubcoreMesh(core_axis_name="c", subcore_axis_name="s")
    fn = pl.kernel(
        functools.partial(_edge_sc_body, head_of_group),
        out_type=[jax.ShapeDtypeStruct((NCORE, NP, F), f32),
                  jax.ShapeDtypeStruct((NCORE, NP, HP), f32)],
        mesh=mesh,
        compiler_params=pltpu.CompilerParams(use_tc_tiling_on_sc=False),
        scratch_types=[
            pltpu.VMEM((CB,), jnp.int32),
            pltpu.VMEM((CB,), jnp.int32),
            pltpu.VMEM((CB, HP), f32),
            pltpu.VMEM((CB, HP), f32),
            pltpu.VMEM((CB, F), f32),
            pltpu.VMEM((2, HP), f32),
            pltpu.VMEM_SHARED((NP, F), f32),
            pltpu.VMEM_SHARED((NP, HP), f32),
        ],
    )
    return fn(ft, elp, erp, elmax, ermax, src, dst, z128, z16)


def kernel(feats, g, W0, al0, ar0, W1, al1, ar1):
    src = g[0]
    dst = g[1]
    alf0 = al0.reshape(1, F)
    arf0 = ar0.reshape(1, F)
    alf1 = al1.reshape(1, F)
    arf1 = ar1.reshape(1, F)

    ft0, elp0, erp0, elm0, erm0 = _dense_stage(feats, W0, alf0, arf0, 8)
    num0, den0 = _edge_stage(ft0, elp0, erp0, elm0, erm0, src, dst, 8)
    h1 = _combine(num0, den0, 8)

    ft1, elp1, erp1, elm1, erm1 = _dense_stage(h1, W1, alf1, arf1, 1)
    num1, den1 = _edge_stage(ft1, elp1, erp1, elm1, erm1, src, dst, 1)
    h_final = _combine(num1, den1, 1)
    return (h1, h_final)
